# Initial kernel scaffold; baseline (speedup 1.0000x reference)
import jax
import jax.numpy as jnp
from jax.experimental import pallas as pl


def kernel(x, router_w, router_b, we_gate, we_up, we_down):
    raise NotImplementedError("write your pallas kernel here")


# speedup vs baseline: 1.6913x; 1.6913x over previous
import jax
import jax.numpy as jnp
from jax import lax
from jax.experimental import pallas as pl
from jax.experimental.pallas import tpu as pltpu
from jax.experimental.pallas import tpu_sc as plsc

E = 16
D = 1024
T = 4096
TOKB = 1024
TM = 128
P = T + E * TM
NT = P // TM
NW = 32
TPW = T // NW


def _router_body(x_ref, rw_ref, rb_ref, e_ref, gr_ref, cnt_ref, carry_ref):
    j = pl.program_id(0)

    @pl.when(j == 0)
    def _init():
        carry_ref[...] = jnp.zeros_like(carry_ref)

    xb = x_ref[...]
    logits = jnp.dot(xb, rw_ref[...], preferred_element_type=jnp.float32)
    logits = logits + rb_ref[0:1, :]
    m = jnp.max(logits, axis=1, keepdims=True)
    lane = lax.broadcasted_iota(jnp.int32, (TOKB, E), 1)
    e_col = jnp.min(jnp.where(logits == m, lane, E), axis=1, keepdims=True)
    onehot = (lane == e_col).astype(jnp.float32)

    tr = lax.broadcasted_iota(jnp.int32, (TOKB, TOKB), 0)
    tc = lax.broadcasted_iota(jnp.int32, (TOKB, TOKB), 1)
    tri = (tr > tc).astype(jnp.float32)
    local_rank = jnp.dot(tri, onehot, preferred_element_type=jnp.float32)
    carry_row = carry_ref[0:1, 0:E]
    grank = jnp.sum((local_rank + carry_row) * onehot, axis=1, keepdims=True)
    new_carry = carry_row + jnp.sum(onehot, axis=0, keepdims=True)
    carry_ref[0:1, 0:E] = new_carry
    cnt_ref[0:1, 0:E] = new_carry

    ps = lax.broadcasted_iota(jnp.int32, (8, TOKB), 0)
    pt = lax.broadcasted_iota(jnp.int32, (8, TOKB), 1)
    pk = ((pt // 128) == ps).astype(jnp.float32)
    ml = lax.broadcasted_iota(jnp.int32, (TOKB, 128), 0)
    ll = lax.broadcasted_iota(jnp.int32, (TOKB, 128), 1)
    bm = ((ml & 127) == ll).astype(jnp.float32)
    e_packed = jnp.dot(pk, e_col.astype(jnp.float32) * bm,
                       preferred_element_type=jnp.float32,
                       precision=lax.Precision.HIGHEST)
    g_packed = jnp.dot(pk, grank * bm, preferred_element_type=jnp.float32,
                       precision=lax.Precision.HIGHEST)
    e_ref[...] = e_packed.astype(jnp.int32)
    gr_ref[...] = g_packed.astype(jnp.int32)


def _router_call(x_flat, rw, rb8):
    nblk = T // TOKB
    return pl.pallas_call(
        _router_body,
        grid=(nblk,),
        in_specs=[
            pl.BlockSpec((TOKB, D), lambda j: (j, 0)),
            pl.BlockSpec((D, E), lambda j: (0, 0)),
            pl.BlockSpec((8, E), lambda j: (0, 0)),
        ],
        out_specs=[
            pl.BlockSpec((8, 128), lambda j: (j, 0)),
            pl.BlockSpec((8, 128), lambda j: (j, 0)),
            pl.BlockSpec((8, 128), lambda j: (0, 0)),
        ],
        out_shape=[
            jax.ShapeDtypeStruct((NW, 128), jnp.int32),
            jax.ShapeDtypeStruct((NW, 128), jnp.int32),
            jax.ShapeDtypeStruct((8, 128), jnp.float32),
        ],
        scratch_shapes=[pltpu.VMEM((8, 128), jnp.float32)],
        compiler_params=pltpu.CompilerParams(
            dimension_semantics=("arbitrary",)),
    )(x_flat, rw, rb8)


def _plan_body(cnt_ref, e_ref, gr_ref, pos_ref, meta_ref):
    c = cnt_ref[0:1, 0:E]
    ci = c.astype(jnp.int32)
    ri = ((ci + (TM - 1)) // TM) * TM
    rf = ri.astype(jnp.float32)

    ea = e_ref[...]
    pos = gr_ref[...]
    jb = lax.broadcasted_iota(jnp.int32, (1, 128), 1) * TM
    te = jnp.full((1, 128), -1, jnp.int32)
    acc = jnp.zeros((1, 1), jnp.float32)
    for k in range(E):
        pk = acc.astype(jnp.int32)
        pos = pos + jnp.where(ea == k, pk, 0)
        te = te + jnp.where(jb >= pk, 1, 0)
        acc = acc + lax.slice(rf, (0, k), (1, k + 1))
    pos_ref[...] = pos
    meta_ref[0:1, :] = te


def _plan_call(cnt, e_p, gr_p):
    return pl.pallas_call(
        _plan_body,
        out_shape=[
            jax.ShapeDtypeStruct((NW, 128), jnp.int32),
            jax.ShapeDtypeStruct((8, 128), jnp.int32),
        ],
    )(cnt, e_p, gr_p)


def _dispatch_body(x_hbm, pos_hbm, out_hbm, buf, idx0, idx1, sem):
    wid = lax.axis_index("c") * 16 + lax.axis_index("s")
    base = wid * TPW
    pltpu.sync_copy(pos_hbm.at[wid, 0], idx0)
    pltpu.sync_copy(pos_hbm.at[wid, 1], idx1)
    pltpu.sync_copy(x_hbm.at[pl.ds(base, 64)], buf)
    pltpu.async_copy(buf, out_hbm.at[idx0], sem).wait()
    pltpu.sync_copy(x_hbm.at[pl.ds(base + 64, 64)], buf)
    pltpu.async_copy(buf, out_hbm.at[idx1], sem).wait()


def _dispatch_call(x_flat, pos3):
    f = pl.kernel(
        _dispatch_body,
        mesh=plsc.VectorSubcoreMesh(core_axis_name="c", subcore_axis_name="s"),
        out_type=jax.ShapeDtypeStruct((P, D), jnp.float32),
        scratch_types=[
            pltpu.VMEM((64, D), jnp.float32),
            pltpu.VMEM((64,), jnp.int32),
            pltpu.VMEM((64,), jnp.int32),
            pltpu.SemaphoreType.DMA,
        ],
    )
    return f(x_flat, pos3)


def _combine_body(ff_hbm, pos_hbm, out_hbm, buf, idx0, idx1, sem):
    wid = lax.axis_index("c") * 16 + lax.axis_index("s")
    base = wid * TPW
    pltpu.sync_copy(pos_hbm.at[wid, 0], idx0)
    pltpu.sync_copy(pos_hbm.at[wid, 1], idx1)
    pltpu.async_copy(ff_hbm.at[idx0], buf, sem).wait()
    pltpu.sync_copy(buf, out_hbm.at[pl.ds(base, 64)])
    pltpu.async_copy(ff_hbm.at[idx1], buf, sem).wait()
    pltpu.sync_copy(buf, out_hbm.at[pl.ds(base + 64, 64)])


def _combine_call(ff, pos3):
    f = pl.kernel(
        _combine_body,
        mesh=plsc.VectorSubcoreMesh(core_axis_name="c", subcore_axis_name="s"),
        out_type=jax.ShapeDtypeStruct((T, D), jnp.float32),
        scratch_types=[
            pltpu.VMEM((64, D), jnp.float32),
            pltpu.VMEM((64,), jnp.int32),
            pltpu.VMEM((64,), jnp.int32),
            pltpu.SemaphoreType.DMA,
        ],
    )
    return f(ff, pos3)


def _ffn_body(te_ref, x_ref, wg_ref, wu_ref, wd_ref, o_ref):
    xb = x_ref[...]
    g = jnp.dot(xb, wg_ref[0], preferred_element_type=jnp.float32)
    u = jnp.dot(xb, wu_ref[0], preferred_element_type=jnp.float32)
    a = g * jax.nn.sigmoid(g) * u
    o_ref[...] = jnp.dot(a, wd_ref[0], preferred_element_type=jnp.float32)


def _ffn_call(tile_e, xs, wg, wu, wd):
    grid_spec = pltpu.PrefetchScalarGridSpec(
        num_scalar_prefetch=1,
        grid=(NT,),
        in_specs=[
            pl.BlockSpec((TM, D), lambda j, te: (j, 0)),
            pl.BlockSpec((1, D, D), lambda j, te: (te[j], 0, 0)),
            pl.BlockSpec((1, D, D), lambda j, te: (te[j], 0, 0)),
            pl.BlockSpec((1, D, D), lambda j, te: (te[j], 0, 0)),
        ],
        out_specs=pl.BlockSpec((TM, D), lambda j, te: (j, 0)),
    )
    return pl.pallas_call(
        _ffn_body,
        grid_spec=grid_spec,
        out_shape=jax.ShapeDtypeStruct((P, D), jnp.float32),
        compiler_params=pltpu.CompilerParams(
            dimension_semantics=("arbitrary",)),
    )(tile_e, xs, wg, wu, wd)


def kernel(x, router_w, router_b, we_gate, we_up, we_down):
    xsh = x.shape
    x_flat = x.reshape(-1, xsh[-1])
    rb8 = jnp.broadcast_to(router_b[None, :], (8, E))
    e_p, gr_p, cnt = _router_call(x_flat, router_w, rb8)
    pos, meta = _plan_call(cnt, e_p, gr_p)
    tile_e = meta[0, :NT]
    pos3 = pos.reshape(NW, 2, 64)
    xs = _dispatch_call(x_flat, pos3)
    ff = _ffn_call(tile_e, xs, we_gate, we_up, we_down)
    out_flat = _combine_call(ff, pos3)
    return out_flat.reshape(xsh)


# speedup vs baseline: 1.7832x; 1.0544x over previous
import jax
import jax.numpy as jnp
from jax import lax
from jax.experimental import pallas as pl
from jax.experimental.pallas import tpu as pltpu
from jax.experimental.pallas import tpu_sc as plsc

E = 16
D = 1024
T = 4096
TOKB = 1024
TM = 128
P = T + E * TM
NT = P // TM
NW = 32
TPW = T // NW
NBLK = T // TOKB


def _router_body(x_ref, rw_ref, rb_ref, xbf_ref, pos_ref, meta_ref,
                 e_s, g_s, carry_ref):
    j = pl.program_id(0)

    @pl.when(j == 0)
    def _init():
        carry_ref[...] = jnp.zeros_like(carry_ref)

    xb = x_ref[...]
    xbf = xb.astype(jnp.bfloat16)
    lo = lax.bitcast_convert_type(xbf[:, :D // 2], jnp.uint16).astype(jnp.uint32)
    hi = lax.bitcast_convert_type(xbf[:, D // 2:], jnp.uint16).astype(jnp.uint32)
    xbf_ref[...] = lax.bitcast_convert_type(lo | (hi << 16), jnp.int32)
    logits = jnp.dot(xb, rw_ref[...], preferred_element_type=jnp.float32)
    logits = logits + rb_ref[0:1, :]
    m = jnp.max(logits, axis=1, keepdims=True)
    lane = lax.broadcasted_iota(jnp.int32, (TOKB, E), 1)
    e_col = jnp.min(jnp.where(logits == m, lane, E), axis=1, keepdims=True)
    onehot = (lane == e_col).astype(jnp.float32)

    tr = lax.broadcasted_iota(jnp.int32, (TOKB, TOKB), 0)
    tc = lax.broadcasted_iota(jnp.int32, (TOKB, TOKB), 1)
    tri = (tr > tc).astype(jnp.float32)
    local_rank = jnp.dot(tri, onehot, preferred_element_type=jnp.float32)
    carry_row = carry_ref[0:1, 0:E]
    grank = jnp.sum((local_rank + carry_row) * onehot, axis=1, keepdims=True)
    new_carry = carry_row + jnp.sum(onehot, axis=0, keepdims=True)
    carry_ref[0:1, 0:E] = new_carry

    ps = lax.broadcasted_iota(jnp.int32, (8, TOKB), 0)
    pt = lax.broadcasted_iota(jnp.int32, (8, TOKB), 1)
    pk = ((pt // 128) == ps).astype(jnp.float32)
    ml = lax.broadcasted_iota(jnp.int32, (TOKB, 128), 0)
    ll = lax.broadcasted_iota(jnp.int32, (TOKB, 128), 1)
    bm = ((ml & 127) == ll).astype(jnp.float32)
    e_packed = jnp.dot(pk, e_col.astype(jnp.float32) * bm,
                       preferred_element_type=jnp.float32,
                       precision=lax.Precision.HIGHEST)
    g_packed = jnp.dot(pk, grank * bm, preferred_element_type=jnp.float32,
                       precision=lax.Precision.HIGHEST)
    rows = TOKB // 128
    e_s[pl.ds(rows * j, rows), :] = e_packed.astype(jnp.int32)
    g_s[pl.ds(rows * j, rows), :] = g_packed.astype(jnp.int32)

    @pl.when(j == NBLK - 1)
    def _plan():
        ci = new_carry.astype(jnp.int32)
        rf = (((ci + (TM - 1)) // TM) * TM).astype(jnp.float32)
        ea = e_s[...]
        pos = g_s[...]
        jb = lax.broadcasted_iota(jnp.int32, (1, 128), 1) * TM
        te = jnp.full((1, 128), -1, jnp.int32)
        acc = jnp.zeros((1, 1), jnp.float32)
        for k in range(E):
            pk_off = acc.astype(jnp.int32)
            pos = pos + jnp.where(ea == k, pk_off, 0)
            te = te + jnp.where(jb >= pk_off, 1, 0)
            acc = acc + lax.slice(rf, (0, k), (1, k + 1))
        pos_ref[...] = pos
        meta_ref[0:1, :] = te


def _router_call(x_flat, rw, rb8):
    return pl.pallas_call(
        _router_body,
        grid=(NBLK,),
        in_specs=[
            pl.BlockSpec((TOKB, D), lambda j: (j, 0)),
            pl.BlockSpec((D, E), lambda j: (0, 0)),
            pl.BlockSpec((8, E), lambda j: (0, 0)),
        ],
        out_specs=[
            pl.BlockSpec((TOKB, D // 2), lambda j: (j, 0)),
            pl.BlockSpec((NW, 128), lambda j: (0, 0)),
            pl.BlockSpec((8, 128), lambda j: (0, 0)),
        ],
        out_shape=[
            jax.ShapeDtypeStruct((T, D // 2), jnp.int32),
            jax.ShapeDtypeStruct((NW, 128), jnp.int32),
            jax.ShapeDtypeStruct((8, 128), jnp.int32),
        ],
        scratch_shapes=[
            pltpu.VMEM((NW, 128), jnp.int32),
            pltpu.VMEM((NW, 128), jnp.int32),
            pltpu.VMEM((8, 128), jnp.float32),
        ],
        compiler_params=pltpu.CompilerParams(
            dimension_semantics=("arbitrary",)),
    )(x_flat, rw, rb8)


def _dispatch_body(x_hbm, pos_hbm, out_hbm, buf, idx, sem):
    wid = lax.axis_index("c") * 16 + lax.axis_index("s")
    base = wid * TPW
    pltpu.sync_copy(pos_hbm.at[wid], idx)
    pltpu.sync_copy(x_hbm.at[pl.ds(base, TPW)], buf)
    pltpu.async_copy(buf, out_hbm.at[idx], sem).wait()


def _dispatch_call(xbf, pos):
    f = pl.kernel(
        _dispatch_body,
        mesh=plsc.VectorSubcoreMesh(core_axis_name="c", subcore_axis_name="s"),
        out_type=jax.ShapeDtypeStruct((P, D // 2), jnp.int32),
        scratch_types=[
            pltpu.VMEM((TPW, D // 2), jnp.int32),
            pltpu.VMEM((TPW,), jnp.int32),
            pltpu.SemaphoreType.DMA,
        ],
    )
    return f(xbf, pos)


def _combine_body(ff_hbm, pos_hbm, out_hbm, buf, idx0, idx1, sem):
    wid = lax.axis_index("c") * 16 + lax.axis_index("s")
    base = wid * TPW
    pltpu.sync_copy(pos_hbm.at[wid, pl.ds(0, 64)], idx0)
    pltpu.sync_copy(pos_hbm.at[wid, pl.ds(64, 64)], idx1)
    pltpu.async_copy(ff_hbm.at[idx0], buf, sem).wait()
    pltpu.sync_copy(buf, out_hbm.at[pl.ds(base, 64)])
    pltpu.async_copy(ff_hbm.at[idx1], buf, sem).wait()
    pltpu.sync_copy(buf, out_hbm.at[pl.ds(base + 64, 64)])


def _combine_call(ff, pos):
    f = pl.kernel(
        _combine_body,
        mesh=plsc.VectorSubcoreMesh(core_axis_name="c", subcore_axis_name="s"),
        out_type=jax.ShapeDtypeStruct((T, D), jnp.float32),
        scratch_types=[
            pltpu.VMEM((64, D), jnp.float32),
            pltpu.VMEM((64,), jnp.int32),
            pltpu.VMEM((64,), jnp.int32),
            pltpu.SemaphoreType.DMA,
        ],
    )
    return f(ff, pos)


def _ffn_body(te_ref, x_ref, wg_ref, wu_ref, wd_ref, o_ref):
    xi = lax.bitcast_convert_type(x_ref[...], jnp.uint32)
    lo = lax.bitcast_convert_type((xi & 0xFFFF).astype(jnp.uint16),
                                  jnp.bfloat16).astype(jnp.float32)
    hi = lax.bitcast_convert_type((xi >> 16).astype(jnp.uint16),
                                  jnp.bfloat16).astype(jnp.float32)
    xb = jnp.concatenate([lo, hi], axis=1)
    g = jnp.dot(xb, wg_ref[0], preferred_element_type=jnp.float32)
    u = jnp.dot(xb, wu_ref[0], preferred_element_type=jnp.float32)
    a = g * jax.nn.sigmoid(g) * u
    o_ref[...] = jnp.dot(a, wd_ref[0], preferred_element_type=jnp.float32)


def _ffn_call(tile_e, xs, wg, wu, wd):
    grid_spec = pltpu.PrefetchScalarGridSpec(
        num_scalar_prefetch=1,
        grid=(NT,),
        in_specs=[
            pl.BlockSpec((TM, D // 2), lambda j, te: (j, 0)),
            pl.BlockSpec((1, D, D), lambda j, te: (te[j], 0, 0)),
            pl.BlockSpec((1, D, D), lambda j, te: (te[j], 0, 0)),
            pl.BlockSpec((1, D, D), lambda j, te: (te[j], 0, 0)),
        ],
        out_specs=pl.BlockSpec((TM, D), lambda j, te: (j, 0)),
    )
    return pl.pallas_call(
        _ffn_body,
        grid_spec=grid_spec,
        out_shape=jax.ShapeDtypeStruct((P, D), jnp.float32),
        compiler_params=pltpu.CompilerParams(
            dimension_semantics=("arbitrary",)),
    )(tile_e, xs, wg, wu, wd)


def kernel(x, router_w, router_b, we_gate, we_up, we_down):
    xsh = x.shape
    x_flat = x.reshape(-1, xsh[-1])
    rb8 = jnp.broadcast_to(router_b[None, :], (8, E))
    xbf, pos, meta = _router_call(x_flat, router_w, rb8)
    tile_e = meta[0, :NT]
    xs = _dispatch_call(xbf, pos)
    ff = _ffn_call(tile_e, xs, we_gate, we_up, we_down)
    out_flat = _combine_call(ff, pos)
    return out_flat.reshape(xsh)
